# Initial kernel scaffold; baseline (speedup 1.0000x reference)
import jax
import jax.numpy as jnp
from jax.experimental import pallas as pl


def kernel(x, time_day, time_week):
    raise NotImplementedError("write your pallas kernel here")


# speedup vs baseline: 6.6885x; 6.6885x over previous
import functools

import jax
import jax.numpy as jnp
from jax.experimental import pallas as pl
from jax.experimental.pallas import tpu as pltpu

_TIME = 288
_WEEK = 8
_F = 64
_NB = 256


def _body(xs1_ref, xs2_ref, tdt_ref, twt_ref, out_ref):
    J = xs1_ref.shape[2]
    didx = (xs1_ref[0] * _TIME).astype(jnp.int32)
    widx = xs2_ref[0].astype(jnp.int32)
    kd = jax.lax.broadcasted_iota(jnp.int32, (_TIME, J), 0)
    kw = jax.lax.broadcasted_iota(jnp.int32, (_WEEK, J), 0)
    ohd = (didx == kd).astype(jnp.bfloat16)
    ohw = (widx == kw).astype(jnp.bfloat16)
    acc = jax.lax.dot_general(
        tdt_ref[...], ohd, (((1,), (0,)), ((), ())),
        preferred_element_type=jnp.float32)
    acc += jax.lax.dot_general(
        twt_ref[...], ohw, (((1,), (0,)), ((), ())),
        preferred_element_type=jnp.float32)
    out_ref[0] = acc


@jax.jit
def kernel(x, time_day, time_week):
    B, T, N, C = x.shape
    F = time_day.shape[1]
    xs1 = jnp.transpose(x[..., 1], (0, 2, 1)).reshape(B, 1, N * T)
    xs2 = jnp.transpose(x[..., 2], (0, 2, 1)).reshape(B, 1, N * T)
    tdt = time_day.T.astype(jnp.bfloat16)
    twt = jnp.pad(time_week, ((0, _WEEK - time_week.shape[0]), (0, 0)))
    twt = twt.T.astype(jnp.bfloat16)

    J = _NB * T
    grid = (B, N // _NB)
    out_flat = pl.pallas_call(
        _body,
        grid=grid,
        in_specs=[
            pl.BlockSpec((1, 1, J), lambda b, n: (b, 0, n)),
            pl.BlockSpec((1, 1, J), lambda b, n: (b, 0, n)),
            pl.BlockSpec((F, _TIME), lambda b, n: (0, 0)),
            pl.BlockSpec((F, _WEEK), lambda b, n: (0, 0)),
        ],
        out_specs=pl.BlockSpec((1, F, J), lambda b, n: (b, 0, n)),
        out_shape=jax.ShapeDtypeStruct((B, F, N * T), jnp.float32),
        compiler_params=pltpu.CompilerParams(
            dimension_semantics=("parallel", "parallel")),
    )(xs1, xs2, tdt, twt)
    return out_flat.reshape(B, F, N, T)
